# baseline (speedup 1.0000x reference)
import functools

import jax
import jax.numpy as jnp
from jax import lax
from jax.experimental import pallas as pl
from jax.experimental.pallas import tpu as pltpu
from jax.experimental.pallas import tpu_sc as plsc

NC, NS, L = 2, 16, 16
NW = NC * NS
SREP = 4


def _body(dim, tz, sz, bw,
          t_tab, s_tab, tweet, sent, out, ttab_v, stab_v, stab2_v, tidx,
          sidx, big, sem, sem2):
    wid = lax.axis_index("s") * NC + lax.axis_index("c")
    row0 = wid * bw
    half = bw // 2
    with jax.named_scope("prologue_dma"):
        cp2 = [
            pltpu.async_copy(tweet.at[pl.ds(row0 + half, half)],
                             tidx.at[pl.ds(half, half)], sem2),
            pltpu.async_copy(sent.at[pl.ds(row0 + half, half)],
                             sidx.at[pl.ds(half, half)], sem2),
        ]
        cps = [
            pltpu.async_copy(t_tab, ttab_v, sem),
            pltpu.async_copy(s_tab, stab_v, sem),
            pltpu.async_copy(tweet.at[pl.ds(row0, half)],
                             tidx.at[pl.ds(0, half)], sem),
            pltpu.async_copy(sent.at[pl.ds(row0, half)],
                             sidx.at[pl.ds(0, half)], sem),
        ]
        for c in cps:
            c.wait()

    lanes = lax.iota(jnp.int32, L)
    zero = jnp.zeros((L,), jnp.float32)
    ncol = (sz + 4) * SREP

    for r in range(dim):
        for k in range(ncol // L):
            stab2_v[r, pl.ds(k * L, L)] = zero
    for s in range(sz):
        for k in range(dim // L):
            v = stab_v[s, pl.ds(k * L, L)]
            for rep in range(SREP):
                plsc.store_scatter(
                    stab2_v,
                    [k * L + lanes, jnp.full((L,), s + 8 * rep, jnp.int32)], v)

    rep_off = (lanes & (SREP - 1)) * 8

    def chunk_body(ch):
        base = pl.multiple_of(ch * L, L)
        t16 = tidx[pl.ds(base, L)]
        s16 = sidx[pl.ds(base, L)]
        m = t16 == 0
        sr = jnp.where(m, sz, s16) + rep_off
        for g in range(0, 2 * dim, L):
            vals = []
            for k in range(L):
                c = g + k
                if c < dim:
                    v = plsc.load_gather(
                        ttab_v, [jnp.full((L,), c, jnp.int32), t16])
                    vals.append(jnp.where(m, 0.0, v))
                else:
                    vals.append(plsc.load_gather(
                        stab2_v, [jnp.full((L,), c - dim, jnp.int32), sr]))
            for k in range(L):
                big[g + k, pl.ds(base, L)] = vals[k]

    with jax.named_scope("gather_loop1"):
        @plsc.parallel_loop(0, half // L)
        def _loop1(ch):
            chunk_body(ch)

    cp1 = pltpu.async_copy(big.at[:, pl.ds(0, half)],
                           out.at[:, pl.ds(row0, half)], sem)
    with jax.named_scope("wait_idx2"):
        for c in cp2:
            c.wait()
    with jax.named_scope("gather_loop2"):
        @plsc.parallel_loop(half // L, bw // L)
        def _loop2(ch):
            chunk_body(ch)

    with jax.named_scope("out_dma"):
        cp1.wait()
        pltpu.sync_copy(big.at[:, pl.ds(half, half)],
                        out.at[:, pl.ds(row0 + half, half)])


def kernel(tweet, sentiment, tweet_table, sentiment_table):
    b = tweet.shape[0]
    dim = tweet_table.shape[1]
    tz = tweet_table.shape[0]
    sz = sentiment_table.shape[0]
    bw = b // NW

    mesh = plsc.VectorSubcoreMesh(core_axis_name="c", subcore_axis_name="s")
    run = pl.kernel(
        functools.partial(_body, dim, tz, sz, bw),
        out_type=jax.ShapeDtypeStruct((2 * dim, b), jnp.float32),
        mesh=mesh,
        scratch_types=[
            pltpu.VMEM((dim, tz), jnp.float32),
            pltpu.VMEM((sz, dim), jnp.float32),
            pltpu.VMEM((dim, (sz + 4) * SREP), jnp.float32),
            pltpu.VMEM((bw,), jnp.int32),
            pltpu.VMEM((bw,), jnp.int32),
            pltpu.VMEM((2 * dim, bw), jnp.float32),
            pltpu.SemaphoreType.DMA,
            pltpu.SemaphoreType.DMA,
        ],
        compiler_params=pltpu.CompilerParams(needs_layout_passes=False),
    )
    out_t = run(tweet_table.T, sentiment_table,
                tweet.astype(jnp.int32), sentiment.astype(jnp.int32))
    return out_t.T
